# baseline (speedup 1.0000x reference)
import functools

import jax
import jax.numpy as jnp
from jax import lax
from jax.experimental import pallas as pl
from jax.experimental.pallas import tpu as pltpu
from jax.experimental.pallas import tpu_sc as plsc

B = 16384
D = 32
NC = 2
NS = 16
NW = NC * NS
BPW = B // NW
CH = 128
NCH = BPW // CH


def _sc_body(sidx_hbm, pidx_hbm, osidx_hbm, opidx_hbm, ws_hbm, wp_hbm,
             out_hbm, sidx_v, pidx_v, osidx_v, opidx_v, srows_v, prows_v,
             gsem, ssem):
    wid = lax.axis_index("s") * NC + lax.axis_index("c")
    row0 = wid * NCH

    pltpu.sync_copy(sidx_hbm.at[pl.ds(row0, NCH)], sidx_v)
    pltpu.sync_copy(pidx_hbm.at[pl.ds(row0, NCH)], pidx_v)
    pltpu.sync_copy(osidx_hbm.at[pl.ds(row0, NCH)], osidx_v)
    pltpu.sync_copy(opidx_hbm.at[pl.ds(row0, NCH)], opidx_v)

    gathers = []
    for j in range(NCH):
        gathers.append(pltpu.async_copy(
            ws_hbm.at[sidx_v.at[j]], srows_v.at[pl.ds(j * CH, CH)], gsem))
    for j in range(NCH):
        gathers.append(pltpu.async_copy(
            wp_hbm.at[pidx_v.at[j]], prows_v.at[pl.ds(j * CH, CH)], gsem))
    for g in gathers:
        g.wait()

    scatters = []
    for j in range(NCH):
        scatters.append(pltpu.async_copy(
            srows_v.at[pl.ds(j * CH, CH)], out_hbm.at[osidx_v.at[j]], ssem))
    for j in range(NCH):
        scatters.append(pltpu.async_copy(
            prows_v.at[pl.ds(j * CH, CH)], out_hbm.at[opidx_v.at[j]], ssem))
    for s in scatters:
        s.wait()


@functools.partial(jax.jit, static_argnames=())
def kernel(item_fea, W_stars, W_postal):
    stars_idx = item_fea[:, 0].reshape(B // CH, CH)
    postal_idx = item_fea[:, 1].reshape(B // CH, CH)
    oidx = jnp.arange(B, dtype=jnp.int32) * 2
    osidx = oidx.reshape(B // CH, CH)
    opidx = (oidx + 1).reshape(B // CH, CH)

    mesh = plsc.VectorSubcoreMesh(core_axis_name="c", subcore_axis_name="s")
    out = pl.kernel(
        _sc_body,
        mesh=mesh,
        out_type=jax.ShapeDtypeStruct((2 * B, D), jnp.float32),
        compiler_params=pltpu.CompilerParams(use_tc_tiling_on_sc=False),
        scratch_types=[
            pltpu.VMEM((NCH, CH), jnp.int32),
            pltpu.VMEM((NCH, CH), jnp.int32),
            pltpu.VMEM((NCH, CH), jnp.int32),
            pltpu.VMEM((NCH, CH), jnp.int32),
            pltpu.VMEM((BPW, D), jnp.float32),
            pltpu.VMEM((BPW, D), jnp.float32),
            pltpu.SemaphoreType.DMA,
            pltpu.SemaphoreType.DMA,
        ],
    )(stars_idx, postal_idx, osidx, opidx, W_stars, W_postal)
    return out.reshape(B, 2 * D)
